# Initial kernel scaffold; baseline (speedup 1.0000x reference)
import jax
import jax.numpy as jnp
from jax.experimental import pallas as pl


def kernel(x, w_conv2, g2, b2, w_conv1, g1, b1, w_proj, gp, bp):
    raise NotImplementedError("write your pallas kernel here")


# speedup vs baseline: 1.3855x; 1.3855x over previous
import functools

import jax
import jax.numpy as jnp
import numpy as np
from jax.experimental import pallas as pl
from jax.experimental.pallas import tpu as pltpu

F32 = jnp.float32


def _conv2d_toeplitz(wk, hh, ww, pad, dil):
    co, ci, kh, kw = wk.shape
    io = np.arange(hh).reshape(hh, 1, 1, 1, 1, 1)
    jo = np.arange(ww).reshape(1, ww, 1, 1, 1, 1)
    ii = np.arange(hh).reshape(1, 1, hh, 1, 1, 1)
    jj = np.arange(ww).reshape(1, 1, 1, ww, 1, 1)
    ka = np.arange(kh).reshape(1, 1, 1, 1, kh, 1)
    kb = np.arange(kw).reshape(1, 1, 1, 1, 1, kw)
    sel = ((ii == io + ka * dil - pad) & (jj == jo + kb * dil - pad)).astype(np.float32)
    m = jnp.einsum("pqijab,ocab->opqcij", jnp.asarray(sel), wk,
                   precision=jax.lax.Precision.HIGHEST)
    return m.reshape(co * hh * ww, ci * hh * ww)


def _bn_affine(rowsum, rowssq, count, gamma, beta, eps):
    mean = rowsum / count
    var = rowssq / count - mean * mean
    inv_std = jax.lax.rsqrt(var + eps)
    scale = gamma * inv_std
    shift = beta - mean * scale
    return scale, shift


def _pass1_kernel(x_ref, w_ref, y_ref, sum_ref, ssq_ref, *, bb):
    y = jnp.dot(w_ref[...], x_ref[...], preferred_element_type=F32)

    @pl.when(pl.program_id(1) == 0)
    def _():
        sum_ref[...] = jnp.zeros_like(sum_ref)
        ssq_ref[...] = jnp.zeros_like(ssq_ref)

    sum_ref[0] += jnp.sum(y, axis=1, keepdims=True)
    ssq_ref[0] += jnp.sum(y * y, axis=1, keepdims=True)

    for bloc in range(bb):
        blk = y[:, bloc * 64:(bloc + 1) * 64].reshape(8, 64, 64)
        t = jnp.swapaxes(blk, 1, 2).reshape(512, 64)
        y_ref[:, bloc * 64:(bloc + 1) * 64] = t


def _pass2_kernel(x_ref, w_ref, sc_ref, sh_ref, y_ref, sum_ref, ssq_ref,
                  gram_ref):
    x = jnp.maximum(x_ref[...] * sc_ref[...] + sh_ref[...], 0.0)
    y = jnp.dot(w_ref[...], x, preferred_element_type=F32)
    y_ref[...] = y

    @pl.when(pl.program_id(1) == 0)
    def _():
        sum_ref[...] = jnp.zeros_like(sum_ref)
        ssq_ref[...] = jnp.zeros_like(ssq_ref)
        gram_ref[...] = jnp.zeros_like(gram_ref)

    sum_ref[0] += jnp.sum(y, axis=1, keepdims=True)
    ssq_ref[0] += jnp.sum(y * y, axis=1, keepdims=True)
    g = jax.lax.dot_general(y, y, (((1,), (1,)), ((), ())),
                            preferred_element_type=F32)
    gram_ref[0] += g


def _pass3_kernel(x_ref, w_ref, c_ref, o_ref):
    o_ref[...] = (jnp.dot(w_ref[...], x_ref[...], preferred_element_type=F32)
                  + c_ref[...])


def kernel(x, w_conv2, g2, b2, w_conv1, g1, b1, w_proj, gp, bp):
    eps = 1e-5
    b, c, u, v, h, w = x.shape
    assert (c, u, v, h, w) == (8, 8, 8, 8, 8) and b % 32 == 0
    x = x.astype(F32)
    n_a = b * h * w
    n_c = n_a * 64

    cores = 2
    bb1 = 16
    tn = bb1 * 64
    nt = n_a // (cores * tn)

    wa_big = _conv2d_toeplitz(w_conv2.astype(F32), u, v, pad=1, dil=1)
    wb_big = _conv2d_toeplitz(w_conv1.astype(F32), h, w, pad=1, dil=1)
    xa = jnp.transpose(x, (1, 2, 3, 0, 4, 5)).reshape(512, n_a)

    ya, s_a, q_a = pl.pallas_call(
        functools.partial(_pass1_kernel, bb=bb1),
        grid=(cores, nt),
        in_specs=[
            pl.BlockSpec((512, tn), lambda ci, i: (0, ci * nt + i)),
            pl.BlockSpec((512, 512), lambda ci, i: (0, 0)),
        ],
        out_specs=(
            pl.BlockSpec((512, tn), lambda ci, i: (0, ci * nt + i)),
            pl.BlockSpec((1, 512, 1), lambda ci, i: (ci, 0, 0)),
            pl.BlockSpec((1, 512, 1), lambda ci, i: (ci, 0, 0)),
        ),
        out_shape=(
            jax.ShapeDtypeStruct((512, n_a), F32),
            jax.ShapeDtypeStruct((cores, 512, 1), F32),
            jax.ShapeDtypeStruct((cores, 512, 1), F32),
        ),
        compiler_params=pltpu.CompilerParams(
            dimension_semantics=("parallel", "arbitrary")),
        cost_estimate=pl.CostEstimate(
            flops=2 * 512 * 512 * n_a, transcendentals=0,
            bytes_accessed=8 * 512 * n_a),
    )(xa, wa_big)

    s_a = jnp.sum(s_a[:, :, 0], axis=0).reshape(c, u * v).sum(axis=1)
    q_a = jnp.sum(q_a[:, :, 0], axis=0).reshape(c, u * v).sum(axis=1)
    scale_a, shift_a = _bn_affine(s_a, q_a, u * v * n_a,
                                  g2.astype(F32), b2.astype(F32), eps)
    sa_rows = jnp.repeat(scale_a, h * w)[:, None]
    ta_rows = jnp.repeat(shift_a, h * w)[:, None]

    yb, s_b, q_b, gram = pl.pallas_call(
        _pass2_kernel,
        grid=(cores, nt),
        in_specs=[
            pl.BlockSpec((512, tn), lambda ci, i: (0, ci * nt + i)),
            pl.BlockSpec((512, 512), lambda ci, i: (0, 0)),
            pl.BlockSpec((512, 1), lambda ci, i: (0, 0)),
            pl.BlockSpec((512, 1), lambda ci, i: (0, 0)),
        ],
        out_specs=(
            pl.BlockSpec((512, tn), lambda ci, i: (0, ci * nt + i)),
            pl.BlockSpec((1, 512, 1), lambda ci, i: (ci, 0, 0)),
            pl.BlockSpec((1, 512, 1), lambda ci, i: (ci, 0, 0)),
            pl.BlockSpec((1, 512, 512), lambda ci, i: (ci, 0, 0)),
        ),
        out_shape=(
            jax.ShapeDtypeStruct((512, n_a), F32),
            jax.ShapeDtypeStruct((cores, 512, 1), F32),
            jax.ShapeDtypeStruct((cores, 512, 1), F32),
            jax.ShapeDtypeStruct((cores, 512, 512), F32),
        ),
        compiler_params=pltpu.CompilerParams(
            dimension_semantics=("parallel", "arbitrary")),
        cost_estimate=pl.CostEstimate(
            flops=4 * 512 * 512 * n_a, transcendentals=0,
            bytes_accessed=8 * 512 * n_a),
    )(ya, wb_big, sa_rows, ta_rows)

    s_b = jnp.sum(s_b[:, :, 0], axis=0)
    q_b = jnp.sum(q_b[:, :, 0], axis=0)
    gram = jnp.sum(gram, axis=0).reshape(c, h * w, c, h * w)
    gram = jnp.einsum("ahbh->ab", gram)
    s_bc = s_b.reshape(c, h * w).sum(axis=1)
    q_bc = q_b.reshape(c, h * w).sum(axis=1)
    scale_b, shift_b = _bn_affine(s_bc, q_bc, h * w * n_a,
                                  g1.astype(F32), b1.astype(F32), eps)

    co = w_proj.shape[0]
    wp2 = w_proj.reshape(co, c).astype(F32)
    wpp = wp2 * scale_b[None, :]
    cst = wp2 @ shift_b
    s_x = s_bc
    s3 = wpp @ s_x + n_c * cst
    q3 = (jnp.einsum("oc,cd,od->o", wpp, gram, wpp)
          + 2.0 * cst * (wpp @ s_x) + n_c * cst * cst)
    scale_c, shift_c = _bn_affine(s3, q3, n_c, gp.astype(F32),
                                  bp.astype(F32), eps)
    wf = scale_c[:, None] * wpp
    cf = scale_c * cst + shift_c

    wf_big = jnp.kron(wf, jnp.eye(h * w, dtype=F32))
    cf_rows = jnp.repeat(cf, h * w)[:, None]

    z = pl.pallas_call(
        _pass3_kernel,
        grid=(cores, nt),
        in_specs=[
            pl.BlockSpec((512, tn), lambda ci, i: (0, ci * nt + i)),
            pl.BlockSpec((co * 64, 512), lambda ci, i: (0, 0)),
            pl.BlockSpec((co * 64, 1), lambda ci, i: (0, 0)),
        ],
        out_specs=pl.BlockSpec((co * 64, tn), lambda ci, i: (0, ci * nt + i)),
        out_shape=jax.ShapeDtypeStruct((co * 64, n_a), F32),
        compiler_params=pltpu.CompilerParams(
            dimension_semantics=("parallel", "arbitrary")),
        cost_estimate=pl.CostEstimate(
            flops=2 * co * 64 * 512 * n_a, transcendentals=0,
            bytes_accessed=4 * (512 * n_a + co * 64 * n_a)),
    )(yb, wf_big, cf_rows)

    return (z.reshape(co, h, w, b, u, v)
            .transpose(3, 0, 4, 5, 1, 2))


# speedup vs baseline: 3.9053x; 2.8188x over previous
import functools

import jax
import jax.numpy as jnp
import numpy as np
from jax.experimental import pallas as pl
from jax.experimental.pallas import tpu as pltpu

F32 = jnp.float32


def _conv2d_toeplitz(wk, hh, ww, pad, dil):
    co, ci, kh, kw = wk.shape
    io = np.arange(hh).reshape(hh, 1, 1, 1, 1, 1)
    jo = np.arange(ww).reshape(1, ww, 1, 1, 1, 1)
    ii = np.arange(hh).reshape(1, 1, hh, 1, 1, 1)
    jj = np.arange(ww).reshape(1, 1, 1, ww, 1, 1)
    ka = np.arange(kh).reshape(1, 1, 1, 1, kh, 1)
    kb = np.arange(kw).reshape(1, 1, 1, 1, 1, kw)
    sel = ((ii == io + ka * dil - pad) & (jj == jo + kb * dil - pad)).astype(np.float32)
    m = jnp.einsum("pqijab,ocab->opqcij", jnp.asarray(sel), wk,
                   precision=jax.lax.Precision.HIGHEST)
    return m.reshape(co * hh * ww, ci * hh * ww)


def _bn_affine(rowsum, rowssq, count, gamma, beta, eps):
    mean = rowsum / count
    var = rowssq / count - mean * mean
    inv_std = jax.lax.rsqrt(var + eps)
    scale = gamma * inv_std
    shift = beta - mean * scale
    return scale, shift


def _pass1_kernel(x_ref, w_ref, y_ref, sum_ref, ssq_ref, *, bb):
    @pl.when(pl.program_id(1) == 0)
    def _():
        sum_ref[...] = jnp.zeros_like(sum_ref)
        ssq_ref[...] = jnp.zeros_like(ssq_ref)

    wmat = w_ref[...]
    s_acc = jnp.zeros((512, 1), F32)
    q_acc = jnp.zeros((512, 1), F32)
    for bloc in range(bb):
        y = jnp.dot(wmat, x_ref[bloc], preferred_element_type=F32)
        s_acc += jnp.sum(y, axis=1, keepdims=True)
        q_acc += jnp.sum(y * y, axis=1, keepdims=True)
        t = jnp.swapaxes(y.reshape(8, 64, 64), 1, 2).reshape(512, 64)
        y_ref[:, bloc * 64:(bloc + 1) * 64] = t
    sum_ref[0] += s_acc
    ssq_ref[0] += q_acc


def _pass2_kernel(x_ref, w_ref, sc_ref, sh_ref, y_ref, sum_ref, ssq_ref,
                  gram_ref):
    x = jnp.maximum(x_ref[...] * sc_ref[...] + sh_ref[...], 0.0)
    y = jnp.dot(w_ref[...], x, preferred_element_type=F32)
    y_ref[...] = y

    @pl.when(pl.program_id(1) == 0)
    def _():
        sum_ref[...] = jnp.zeros_like(sum_ref)
        ssq_ref[...] = jnp.zeros_like(ssq_ref)
        gram_ref[...] = jnp.zeros_like(gram_ref)

    sum_ref[0] += jnp.sum(y, axis=1, keepdims=True)
    ssq_ref[0] += jnp.sum(y * y, axis=1, keepdims=True)
    g = jax.lax.dot_general(y, y, (((1,), (1,)), ((), ())),
                            preferred_element_type=F32)
    gram_ref[0] += g


def _pass3_kernel(x_ref, w_ref, c_ref, o_ref, *, bb):
    flats = []
    for bloc in range(bb):
        piece = x_ref[:, bloc * 64:(bloc + 1) * 64]
        sw = jnp.swapaxes(piece.reshape(8, 64, 64), 1, 2)
        flats.append(sw.reshape(8, 4096))
    t = jnp.concatenate(flats, axis=0)
    o_ref[...] = (jnp.dot(w_ref[...], t, preferred_element_type=F32)
                  + c_ref[...])


def kernel(x, w_conv2, g2, b2, w_conv1, g1, b1, w_proj, gp, bp):
    eps = 1e-5
    b, c, u, v, h, w = x.shape
    assert (c, u, v, h, w) == (8, 8, 8, 8, 8) and b % 32 == 0
    x = x.astype(F32)
    n_a = b * h * w
    n_c = n_a * 64

    cores = 2
    bb1 = 16
    tn = bb1 * 64
    nt = n_a // (cores * tn)

    wa_big = _conv2d_toeplitz(w_conv2.astype(F32), u, v, pad=1, dil=1)
    wb_big = _conv2d_toeplitz(w_conv1.astype(F32), h, w, pad=1, dil=1)
    x3 = x.reshape(b, 512, 64)

    ya, s_a, q_a = pl.pallas_call(
        functools.partial(_pass1_kernel, bb=bb1),
        grid=(cores, nt),
        in_specs=[
            pl.BlockSpec((bb1, 512, 64), lambda ci, i: (ci * nt + i, 0, 0)),
            pl.BlockSpec((512, 512), lambda ci, i: (0, 0)),
        ],
        out_specs=(
            pl.BlockSpec((512, tn), lambda ci, i: (0, ci * nt + i)),
            pl.BlockSpec((1, 512, 1), lambda ci, i: (ci, 0, 0)),
            pl.BlockSpec((1, 512, 1), lambda ci, i: (ci, 0, 0)),
        ),
        out_shape=(
            jax.ShapeDtypeStruct((512, n_a), F32),
            jax.ShapeDtypeStruct((cores, 512, 1), F32),
            jax.ShapeDtypeStruct((cores, 512, 1), F32),
        ),
        compiler_params=pltpu.CompilerParams(
            dimension_semantics=("parallel", "arbitrary")),
        cost_estimate=pl.CostEstimate(
            flops=2 * 512 * 512 * n_a, transcendentals=0,
            bytes_accessed=8 * 512 * n_a),
    )(x3, wa_big)

    s_a = jnp.sum(s_a[:, :, 0], axis=0).reshape(c, u * v).sum(axis=1)
    q_a = jnp.sum(q_a[:, :, 0], axis=0).reshape(c, u * v).sum(axis=1)
    scale_a, shift_a = _bn_affine(s_a, q_a, u * v * n_a,
                                  g2.astype(F32), b2.astype(F32), eps)
    sa_rows = jnp.repeat(scale_a, h * w)[:, None]
    ta_rows = jnp.repeat(shift_a, h * w)[:, None]

    yb, s_b, q_b, gram = pl.pallas_call(
        _pass2_kernel,
        grid=(cores, nt),
        in_specs=[
            pl.BlockSpec((512, tn), lambda ci, i: (0, ci * nt + i)),
            pl.BlockSpec((512, 512), lambda ci, i: (0, 0)),
            pl.BlockSpec((512, 1), lambda ci, i: (0, 0)),
            pl.BlockSpec((512, 1), lambda ci, i: (0, 0)),
        ],
        out_specs=(
            pl.BlockSpec((512, tn), lambda ci, i: (0, ci * nt + i)),
            pl.BlockSpec((1, 512, 1), lambda ci, i: (ci, 0, 0)),
            pl.BlockSpec((1, 512, 1), lambda ci, i: (ci, 0, 0)),
            pl.BlockSpec((1, 512, 512), lambda ci, i: (ci, 0, 0)),
        ),
        out_shape=(
            jax.ShapeDtypeStruct((512, n_a), F32),
            jax.ShapeDtypeStruct((cores, 512, 1), F32),
            jax.ShapeDtypeStruct((cores, 512, 1), F32),
            jax.ShapeDtypeStruct((cores, 512, 512), F32),
        ),
        compiler_params=pltpu.CompilerParams(
            dimension_semantics=("parallel", "arbitrary")),
        cost_estimate=pl.CostEstimate(
            flops=4 * 512 * 512 * n_a, transcendentals=0,
            bytes_accessed=8 * 512 * n_a),
    )(ya, wb_big, sa_rows, ta_rows)

    s_b = jnp.sum(s_b[:, :, 0], axis=0)
    q_b = jnp.sum(q_b[:, :, 0], axis=0)
    gram = jnp.sum(gram, axis=0).reshape(c, h * w, c, h * w)
    gram = jnp.einsum("ahbh->ab", gram)
    s_bc = s_b.reshape(c, h * w).sum(axis=1)
    q_bc = q_b.reshape(c, h * w).sum(axis=1)
    scale_b, shift_b = _bn_affine(s_bc, q_bc, h * w * n_a,
                                  g1.astype(F32), b1.astype(F32), eps)

    co = w_proj.shape[0]
    wp2 = w_proj.reshape(co, c).astype(F32)
    wpp = wp2 * scale_b[None, :]
    cst = wp2 @ shift_b
    s_x = s_bc
    s3 = wpp @ s_x + n_c * cst
    q3 = (jnp.einsum("oc,cd,od->o", wpp, gram, wpp)
          + 2.0 * cst * (wpp @ s_x) + n_c * cst * cst)
    scale_c, shift_c = _bn_affine(s3, q3, n_c, gp.astype(F32),
                                  bp.astype(F32), eps)
    wf = scale_c[:, None] * wpp
    cf = scale_c * cst + shift_c

    bb3 = 16
    nt3 = b // (cores * bb3)
    wf_bd = jnp.kron(jnp.eye(bb3, dtype=F32), wf)
    cf_bd = jnp.tile(cf, bb3)[:, None]

    out2 = pl.pallas_call(
        functools.partial(_pass3_kernel, bb=bb3),
        grid=(cores, nt3),
        in_specs=[
            pl.BlockSpec((512, bb3 * 64), lambda ci, i: (0, ci * nt3 + i)),
            pl.BlockSpec((bb3 * co, bb3 * c), lambda ci, i: (0, 0)),
            pl.BlockSpec((bb3 * co, 1), lambda ci, i: (0, 0)),
        ],
        out_specs=pl.BlockSpec((bb3 * co, 4096),
                               lambda ci, i: (ci * nt3 + i, 0)),
        out_shape=jax.ShapeDtypeStruct((b * co, 4096), F32),
        compiler_params=pltpu.CompilerParams(
            dimension_semantics=("parallel", "arbitrary")),
        cost_estimate=pl.CostEstimate(
            flops=2 * co * c * 4096 * b, transcendentals=0,
            bytes_accessed=4 * (512 * n_a + b * co * 4096)),
    )(yb, wf_bd, cf_bd)

    return out2.reshape(b, co, u, v, h, w)


# speedup vs baseline: 4.0312x; 1.0322x over previous
import functools

import jax
import jax.numpy as jnp
import numpy as np
from jax.experimental import pallas as pl
from jax.experimental.pallas import tpu as pltpu

F32 = jnp.float32


def _conv2d_toeplitz(wk, hh, ww, pad, dil):
    co, ci, kh, kw = wk.shape
    io = np.arange(hh).reshape(hh, 1, 1, 1, 1, 1)
    jo = np.arange(ww).reshape(1, ww, 1, 1, 1, 1)
    ii = np.arange(hh).reshape(1, 1, hh, 1, 1, 1)
    jj = np.arange(ww).reshape(1, 1, 1, ww, 1, 1)
    ka = np.arange(kh).reshape(1, 1, 1, 1, kh, 1)
    kb = np.arange(kw).reshape(1, 1, 1, 1, 1, kw)
    sel = ((ii == io + ka * dil - pad) & (jj == jo + kb * dil - pad)).astype(np.float32)
    m = jnp.einsum("pqijab,ocab->opqcij", jnp.asarray(sel), wk,
                   precision=jax.lax.Precision.HIGHEST)
    return m.reshape(co * hh * ww, ci * hh * ww)


def _bn_affine(rowsum, rowssq, count, gamma, beta, eps):
    mean = rowsum / count
    var = rowssq / count - mean * mean
    inv_std = jax.lax.rsqrt(var + eps)
    scale = gamma * inv_std
    shift = beta - mean * scale
    return scale, shift


def _pass1_kernel(x_ref, w_ref, y_ref, sum_ref, ssq_ref, *, bb):
    @pl.when(pl.program_id(1) == 0)
    def _():
        sum_ref[...] = jnp.zeros_like(sum_ref)
        ssq_ref[...] = jnp.zeros_like(ssq_ref)

    wmat = w_ref[...]
    s_acc = jnp.zeros((512, 1), F32)
    q_acc = jnp.zeros((512, 1), F32)
    for bloc in range(bb):
        y = jnp.dot(wmat, x_ref[bloc], preferred_element_type=F32)
        s_acc += jnp.sum(y, axis=1, keepdims=True)
        q_acc += jnp.sum(y * y, axis=1, keepdims=True)
        t = jnp.swapaxes(y.reshape(8, 64, 64), 1, 2).reshape(512, 64)
        y_ref[:, bloc * 64:(bloc + 1) * 64] = t.astype(jnp.bfloat16)
    sum_ref[0] += s_acc
    ssq_ref[0] += q_acc


def _pass2_kernel(x_ref, w_ref, sc_ref, sh_ref, y_ref, sum_ref, ssq_ref,
                  gram_ref):
    x = jnp.maximum(x_ref[...].astype(jnp.float32) * sc_ref[...]
                    + sh_ref[...], 0.0)
    y = jnp.dot(w_ref[...], x, preferred_element_type=F32)
    y_ref[...] = y.astype(jnp.bfloat16)

    @pl.when(pl.program_id(1) == 0)
    def _():
        sum_ref[...] = jnp.zeros_like(sum_ref)
        ssq_ref[...] = jnp.zeros_like(ssq_ref)
        gram_ref[...] = jnp.zeros_like(gram_ref)

    sum_ref[0] += jnp.sum(y, axis=1, keepdims=True)
    ssq_ref[0] += jnp.sum(y * y, axis=1, keepdims=True)
    g = jax.lax.dot_general(y, y, (((1,), (1,)), ((), ())),
                            preferred_element_type=F32)
    gram_ref[0] += g


def _pass3_kernel(x_ref, w_ref, c_ref, o_ref, *, bb):
    flats = []
    for bloc in range(bb):
        piece = x_ref[:, bloc * 64:(bloc + 1) * 64]
        sw = jnp.swapaxes(piece.reshape(8, 64, 64), 1, 2)
        flats.append(sw.reshape(8, 4096))
    t = jnp.concatenate(flats, axis=0).astype(jnp.float32)
    o_ref[...] = (jnp.dot(w_ref[...], t, preferred_element_type=F32)
                  + c_ref[...])


def kernel(x, w_conv2, g2, b2, w_conv1, g1, b1, w_proj, gp, bp):
    eps = 1e-5
    b, c, u, v, h, w = x.shape
    assert (c, u, v, h, w) == (8, 8, 8, 8, 8) and b % 32 == 0
    x = x.astype(F32)
    n_a = b * h * w
    n_c = n_a * 64

    cores = 2
    bb1 = 16
    tn = bb1 * 64
    nt = n_a // (cores * tn)

    wa_big = _conv2d_toeplitz(w_conv2.astype(F32), u, v, pad=1, dil=1)
    wb_big = _conv2d_toeplitz(w_conv1.astype(F32), h, w, pad=1, dil=1)
    x3 = x.reshape(b, 512, 64)

    ya, s_a, q_a = pl.pallas_call(
        functools.partial(_pass1_kernel, bb=bb1),
        grid=(cores, nt),
        in_specs=[
            pl.BlockSpec((bb1, 512, 64), lambda ci, i: (ci * nt + i, 0, 0)),
            pl.BlockSpec((512, 512), lambda ci, i: (0, 0)),
        ],
        out_specs=(
            pl.BlockSpec((512, tn), lambda ci, i: (0, ci * nt + i)),
            pl.BlockSpec((1, 512, 1), lambda ci, i: (ci, 0, 0)),
            pl.BlockSpec((1, 512, 1), lambda ci, i: (ci, 0, 0)),
        ),
        out_shape=(
            jax.ShapeDtypeStruct((512, n_a), jnp.bfloat16),
            jax.ShapeDtypeStruct((cores, 512, 1), F32),
            jax.ShapeDtypeStruct((cores, 512, 1), F32),
        ),
        compiler_params=pltpu.CompilerParams(
            dimension_semantics=("parallel", "arbitrary")),
        cost_estimate=pl.CostEstimate(
            flops=2 * 512 * 512 * n_a, transcendentals=0,
            bytes_accessed=8 * 512 * n_a),
    )(x3, wa_big)

    s_a = jnp.sum(s_a[:, :, 0], axis=0).reshape(c, u * v).sum(axis=1)
    q_a = jnp.sum(q_a[:, :, 0], axis=0).reshape(c, u * v).sum(axis=1)
    scale_a, shift_a = _bn_affine(s_a, q_a, u * v * n_a,
                                  g2.astype(F32), b2.astype(F32), eps)
    sa_rows = jnp.repeat(scale_a, h * w)[:, None]
    ta_rows = jnp.repeat(shift_a, h * w)[:, None]

    yb, s_b, q_b, gram = pl.pallas_call(
        _pass2_kernel,
        grid=(cores, nt),
        in_specs=[
            pl.BlockSpec((512, tn), lambda ci, i: (0, ci * nt + i)),
            pl.BlockSpec((512, 512), lambda ci, i: (0, 0)),
            pl.BlockSpec((512, 1), lambda ci, i: (0, 0)),
            pl.BlockSpec((512, 1), lambda ci, i: (0, 0)),
        ],
        out_specs=(
            pl.BlockSpec((512, tn), lambda ci, i: (0, ci * nt + i)),
            pl.BlockSpec((1, 512, 1), lambda ci, i: (ci, 0, 0)),
            pl.BlockSpec((1, 512, 1), lambda ci, i: (ci, 0, 0)),
            pl.BlockSpec((1, 512, 512), lambda ci, i: (ci, 0, 0)),
        ),
        out_shape=(
            jax.ShapeDtypeStruct((512, n_a), jnp.bfloat16),
            jax.ShapeDtypeStruct((cores, 512, 1), F32),
            jax.ShapeDtypeStruct((cores, 512, 1), F32),
            jax.ShapeDtypeStruct((cores, 512, 512), F32),
        ),
        compiler_params=pltpu.CompilerParams(
            dimension_semantics=("parallel", "arbitrary")),
        cost_estimate=pl.CostEstimate(
            flops=4 * 512 * 512 * n_a, transcendentals=0,
            bytes_accessed=8 * 512 * n_a),
    )(ya, wb_big, sa_rows, ta_rows)

    s_b = jnp.sum(s_b[:, :, 0], axis=0)
    q_b = jnp.sum(q_b[:, :, 0], axis=0)
    gram = jnp.sum(gram, axis=0).reshape(c, h * w, c, h * w)
    gram = jnp.einsum("ahbh->ab", gram)
    s_bc = s_b.reshape(c, h * w).sum(axis=1)
    q_bc = q_b.reshape(c, h * w).sum(axis=1)
    scale_b, shift_b = _bn_affine(s_bc, q_bc, h * w * n_a,
                                  g1.astype(F32), b1.astype(F32), eps)

    co = w_proj.shape[0]
    wp2 = w_proj.reshape(co, c).astype(F32)
    wpp = wp2 * scale_b[None, :]
    cst = wp2 @ shift_b
    s_x = s_bc
    s3 = wpp @ s_x + n_c * cst
    q3 = (jnp.einsum("oc,cd,od->o", wpp, gram, wpp)
          + 2.0 * cst * (wpp @ s_x) + n_c * cst * cst)
    scale_c, shift_c = _bn_affine(s3, q3, n_c, gp.astype(F32),
                                  bp.astype(F32), eps)
    wf = scale_c[:, None] * wpp
    cf = scale_c * cst + shift_c

    bb3 = 16
    nt3 = b // (cores * bb3)
    wf_bd = jnp.kron(jnp.eye(bb3, dtype=F32), wf)
    cf_bd = jnp.tile(cf, bb3)[:, None]

    out2 = pl.pallas_call(
        functools.partial(_pass3_kernel, bb=bb3),
        grid=(cores, nt3),
        in_specs=[
            pl.BlockSpec((512, bb3 * 64), lambda ci, i: (0, ci * nt3 + i)),
            pl.BlockSpec((bb3 * co, bb3 * c), lambda ci, i: (0, 0)),
            pl.BlockSpec((bb3 * co, 1), lambda ci, i: (0, 0)),
        ],
        out_specs=pl.BlockSpec((bb3 * co, 4096),
                               lambda ci, i: (ci * nt3 + i, 0)),
        out_shape=jax.ShapeDtypeStruct((b * co, 4096), F32),
        compiler_params=pltpu.CompilerParams(
            dimension_semantics=("parallel", "arbitrary")),
        cost_estimate=pl.CostEstimate(
            flops=2 * co * c * 4096 * b, transcendentals=0,
            bytes_accessed=4 * (512 * n_a + b * co * 4096)),
    )(yb, wf_bd, cf_bd)

    return out2.reshape(b, co, u, v, h, w)


# speedup vs baseline: 4.0316x; 1.0001x over previous
import functools

import jax
import jax.numpy as jnp
import numpy as np
from jax.experimental import pallas as pl
from jax.experimental.pallas import tpu as pltpu

F32 = jnp.float32


def _conv2d_toeplitz(wk, hh, ww, pad, dil):
    co, ci, kh, kw = wk.shape
    io = np.arange(hh).reshape(hh, 1, 1, 1, 1, 1)
    jo = np.arange(ww).reshape(1, ww, 1, 1, 1, 1)
    ii = np.arange(hh).reshape(1, 1, hh, 1, 1, 1)
    jj = np.arange(ww).reshape(1, 1, 1, ww, 1, 1)
    ka = np.arange(kh).reshape(1, 1, 1, 1, kh, 1)
    kb = np.arange(kw).reshape(1, 1, 1, 1, 1, kw)
    sel = ((ii == io + ka * dil - pad) & (jj == jo + kb * dil - pad)).astype(np.float32)
    m = jnp.einsum("pqijab,ocab->opqcij", jnp.asarray(sel), wk,
                   precision=jax.lax.Precision.HIGHEST)
    return m.reshape(co * hh * ww, ci * hh * ww)


def _bn_affine(rowsum, rowssq, count, gamma, beta, eps):
    mean = rowsum / count
    var = rowssq / count - mean * mean
    inv_std = jax.lax.rsqrt(var + eps)
    scale = gamma * inv_std
    shift = beta - mean * scale
    return scale, shift


def _pass1_kernel(x_ref, w_ref, y_ref, sum_ref, ssq_ref, *, bb):
    @pl.when(pl.program_id(1) == 0)
    def _():
        sum_ref[...] = jnp.zeros_like(sum_ref)
        ssq_ref[...] = jnp.zeros_like(ssq_ref)

    wmat = w_ref[...]
    s_acc = jnp.zeros((512, 1), F32)
    q_acc = jnp.zeros((512, 1), F32)
    for bloc in range(bb):
        y = jnp.dot(wmat, x_ref[bloc], preferred_element_type=F32)
        s_acc += jnp.sum(y, axis=1, keepdims=True)
        q_acc += jnp.sum(y * y, axis=1, keepdims=True)
        t = jnp.swapaxes(y.reshape(8, 64, 64), 1, 2).reshape(512, 64)
        y_ref[:, bloc * 64:(bloc + 1) * 64] = t.astype(jnp.bfloat16)
    sum_ref[0] += s_acc
    ssq_ref[0] += q_acc


def _pass2_kernel(x_ref, w_ref, sc_ref, sh_ref, y_ref, sum_ref, ssq_ref,
                  gram_ref):
    x = jnp.maximum(x_ref[...].astype(jnp.float32) * sc_ref[...]
                    + sh_ref[...], 0.0)
    y = jnp.dot(w_ref[...], x, preferred_element_type=F32)
    y_ref[...] = y.astype(jnp.bfloat16)

    @pl.when(pl.program_id(1) == 0)
    def _():
        sum_ref[...] = jnp.zeros_like(sum_ref)
        ssq_ref[...] = jnp.zeros_like(ssq_ref)
        gram_ref[...] = jnp.zeros_like(gram_ref)

    sum_ref[0] += jnp.sum(y, axis=1, keepdims=True)
    ssq_ref[0] += jnp.sum(y * y, axis=1, keepdims=True)
    g = jax.lax.dot_general(y, y, (((1,), (1,)), ((), ())),
                            preferred_element_type=F32)
    gram_ref[0] += g


def _pass3_kernel(x_ref, w_ref, c_ref, o_ref, *, bb):
    flats = []
    for bloc in range(bb):
        piece = x_ref[:, bloc * 64:(bloc + 1) * 64]
        sw = jnp.swapaxes(piece.reshape(8, 64, 64), 1, 2)
        flats.append(sw.reshape(8, 4096))
    t = jnp.concatenate(flats, axis=0).astype(jnp.float32)
    o_ref[...] = (jnp.dot(w_ref[...], t, preferred_element_type=F32)
                  + c_ref[...])


def kernel(x, w_conv2, g2, b2, w_conv1, g1, b1, w_proj, gp, bp):
    eps = 1e-5
    b, c, u, v, h, w = x.shape
    assert (c, u, v, h, w) == (8, 8, 8, 8, 8) and b % 32 == 0
    x = x.astype(F32)
    n_a = b * h * w
    n_c = n_a * 64

    cores = 2
    bb1 = 16
    tn = bb1 * 64
    nt = n_a // (cores * tn)

    wa_big = _conv2d_toeplitz(w_conv2.astype(F32), u, v, pad=1, dil=1)
    wb_big = _conv2d_toeplitz(w_conv1.astype(F32), h, w, pad=1, dil=1)
    x3 = x.reshape(b, 512, 64)

    ya, s_a, q_a = pl.pallas_call(
        functools.partial(_pass1_kernel, bb=bb1),
        grid=(cores, nt),
        in_specs=[
            pl.BlockSpec((bb1, 512, 64), lambda ci, i: (ci * nt + i, 0, 0)),
            pl.BlockSpec((512, 512), lambda ci, i: (0, 0)),
        ],
        out_specs=(
            pl.BlockSpec((512, tn), lambda ci, i: (0, ci * nt + i)),
            pl.BlockSpec((1, 512, 1), lambda ci, i: (ci, 0, 0)),
            pl.BlockSpec((1, 512, 1), lambda ci, i: (ci, 0, 0)),
        ),
        out_shape=(
            jax.ShapeDtypeStruct((512, n_a), jnp.bfloat16),
            jax.ShapeDtypeStruct((cores, 512, 1), F32),
            jax.ShapeDtypeStruct((cores, 512, 1), F32),
        ),
        compiler_params=pltpu.CompilerParams(
            dimension_semantics=("parallel", "arbitrary")),
        cost_estimate=pl.CostEstimate(
            flops=2 * 512 * 512 * n_a, transcendentals=0,
            bytes_accessed=8 * 512 * n_a),
    )(x3, wa_big)

    s_a = jnp.sum(s_a[:, :, 0], axis=0).reshape(c, u * v).sum(axis=1)
    q_a = jnp.sum(q_a[:, :, 0], axis=0).reshape(c, u * v).sum(axis=1)
    scale_a, shift_a = _bn_affine(s_a, q_a, u * v * n_a,
                                  g2.astype(F32), b2.astype(F32), eps)
    sa_rows = jnp.repeat(scale_a, h * w)[:, None]
    ta_rows = jnp.repeat(shift_a, h * w)[:, None]

    yb, s_b, q_b, gram = pl.pallas_call(
        _pass2_kernel,
        grid=(cores, nt),
        in_specs=[
            pl.BlockSpec((512, tn), lambda ci, i: (0, ci * nt + i)),
            pl.BlockSpec((512, 512), lambda ci, i: (0, 0)),
            pl.BlockSpec((512, 1), lambda ci, i: (0, 0)),
            pl.BlockSpec((512, 1), lambda ci, i: (0, 0)),
        ],
        out_specs=(
            pl.BlockSpec((512, tn), lambda ci, i: (0, ci * nt + i)),
            pl.BlockSpec((1, 512, 1), lambda ci, i: (ci, 0, 0)),
            pl.BlockSpec((1, 512, 1), lambda ci, i: (ci, 0, 0)),
            pl.BlockSpec((1, 512, 512), lambda ci, i: (ci, 0, 0)),
        ),
        out_shape=(
            jax.ShapeDtypeStruct((512, n_a), jnp.bfloat16),
            jax.ShapeDtypeStruct((cores, 512, 1), F32),
            jax.ShapeDtypeStruct((cores, 512, 1), F32),
            jax.ShapeDtypeStruct((cores, 512, 512), F32),
        ),
        compiler_params=pltpu.CompilerParams(
            dimension_semantics=("parallel", "arbitrary")),
        cost_estimate=pl.CostEstimate(
            flops=4 * 512 * 512 * n_a, transcendentals=0,
            bytes_accessed=8 * 512 * n_a),
    )(ya, wb_big, sa_rows, ta_rows)

    s_b = jnp.sum(s_b[:, :, 0], axis=0)
    q_b = jnp.sum(q_b[:, :, 0], axis=0)
    zsum = jnp.sum(gram, axis=0)
    hwmask = jnp.tile(jnp.eye(h * w, dtype=F32), (c, c))
    gram = (zsum * hwmask).reshape(c, h * w, c * h * w).sum(axis=1)
    gram = gram.reshape(c, c, h * w).sum(axis=2)
    s_bc = s_b.reshape(c, h * w).sum(axis=1)
    q_bc = q_b.reshape(c, h * w).sum(axis=1)
    scale_b, shift_b = _bn_affine(s_bc, q_bc, h * w * n_a,
                                  g1.astype(F32), b1.astype(F32), eps)

    co = w_proj.shape[0]
    wp2 = w_proj.reshape(co, c).astype(F32)
    wpp = wp2 * scale_b[None, :]
    cst = wp2 @ shift_b
    s_x = s_bc
    s3 = wpp @ s_x + n_c * cst
    q3 = (jnp.einsum("oc,cd,od->o", wpp, gram, wpp)
          + 2.0 * cst * (wpp @ s_x) + n_c * cst * cst)
    scale_c, shift_c = _bn_affine(s3, q3, n_c, gp.astype(F32),
                                  bp.astype(F32), eps)
    wf = scale_c[:, None] * wpp
    cf = scale_c * cst + shift_c

    bb3 = 16
    nt3 = b // (cores * bb3)
    wf_bd = jnp.kron(jnp.eye(bb3, dtype=F32), wf)
    cf_bd = jnp.tile(cf, bb3)[:, None]

    out2 = pl.pallas_call(
        functools.partial(_pass3_kernel, bb=bb3),
        grid=(cores, nt3),
        in_specs=[
            pl.BlockSpec((512, bb3 * 64), lambda ci, i: (0, ci * nt3 + i)),
            pl.BlockSpec((bb3 * co, bb3 * c), lambda ci, i: (0, 0)),
            pl.BlockSpec((bb3 * co, 1), lambda ci, i: (0, 0)),
        ],
        out_specs=pl.BlockSpec((bb3 * co, 4096),
                               lambda ci, i: (ci * nt3 + i, 0)),
        out_shape=jax.ShapeDtypeStruct((b * co, 4096), F32),
        compiler_params=pltpu.CompilerParams(
            dimension_semantics=("parallel", "arbitrary")),
        cost_estimate=pl.CostEstimate(
            flops=2 * co * c * 4096 * b, transcendentals=0,
            bytes_accessed=4 * (512 * n_a + b * co * 4096)),
    )(yb, wf_bd, cf_bd)

    return out2.reshape(b, co, u, v, h, w)
